# baseline (speedup 1.0000x reference)
import functools

import jax
import jax.numpy as jnp
from jax import lax
from jax.experimental import pallas as pl
from jax.experimental.pallas import tpu as pltpu
from jax.experimental.pallas import tpu_sc as plsc

NC = 2
NS = 16
NW = NC * NS
LANES = 16
EPS = 1e-5
BB = 128
STRIDE = 129
QOFF = 64 * STRIDE


def _rsqrt(v):
    i = lax.bitcast_convert_type(v, jnp.int32)
    i = jnp.int32(0x5F3759DF) - lax.shift_right_logical(i, 1)
    y = lax.bitcast_convert_type(i, jnp.float32)
    for _ in range(3):
        y = y * (1.5 - 0.5 * v * y * y)
    return y


def _make_sc_kernel(B, L, D):
    T = B * L
    n_blocks = T // BB
    bpw = n_blocks // NW
    nd2 = 2 * D // LANES
    ncol = BB // LANES
    dt_n = D // 8
    bt_n = B // BB

    mesh = plsc.VectorSubcoreMesh(core_axis_name="c", subcore_axis_name="s")

    @functools.partial(
        pl.kernel,
        out_type=jax.ShapeDtypeStruct((L, dt_n, bt_n, 8, BB), jnp.float32),
        mesh=mesh,
        compiler_params=pltpu.CompilerParams(
            needs_layout_passes=False, use_tc_tiling_on_sc=True),
        scratch_types=[
            pltpu.VMEM((16, B), jnp.int32),
            pltpu.VMEM((16, D // 8, BB), jnp.float32),
            pltpu.VMEM((2, D // 8, BB), jnp.float32),
            pltpu.VMEM((BB, 2 * D), jnp.float32),
            pltpu.VMEM((BB, 2 * D), jnp.float32),
            pltpu.VMEM((BB,), jnp.int32),
            pltpu.VMEM((BB,), jnp.int32),
            pltpu.VMEM((2 * D * STRIDE,), jnp.float32),
            pltpu.VMEM((D, BB), jnp.float32),
            pltpu.VMEM((D, BB), jnp.float32),
            pltpu.SemaphoreType.DMA,
            pltpu.SemaphoreType.DMA,
            pltpu.SemaphoreType.DMA,
            pltpu.SemaphoreType.DMA,
        ],
    )
    def sc_kernel(ids_hbm, pos_hbm, gb_hbm, table_hbm, out_hbm,
                  ids_v, pos_v, gb_v, g0_v, g1_v, x0_v, x1_v, tt_v,
                  s0_v, s1_v, gsem0, gsem1, ssem0, ssem1):
        wid = lax.axis_index("s") * NC + lax.axis_index("c")
        g_base = wid * bpw
        t0 = (g_base // bt_n) // 8 * 8

        pltpu.sync_copy(ids_hbm.at[pl.ds(t0, 16)], ids_v)
        pltpu.sync_copy(pos_hbm.at[pl.ds(t0, 16)], pos_v)
        pltpu.sync_copy(gb_hbm, gb_v)

        lane = lax.iota(jnp.int32, LANES)
        inv_d = jnp.float32(1.0 / D)
        sc_idx = [(lane + dq * LANES) * jnp.int32(STRIDE)
                  for dq in range(nd2)]

        gbufs = [(g0_v, gsem0), (g1_v, gsem1)]
        xbufs = [x0_v, x1_v]
        sbufs = [(s0_v, ssem0), (s1_v, ssem1)]

        def block_lbt(i):
            g = g_base + i
            return g // bt_n, g % bt_n

        def gather_start(i, gbuf, gsem, xbuf):
            l, bt = block_lbt(i)
            lr = l - t0
            for c in range(ncol):
                ids16 = ids_v[lr, pl.ds(bt * BB + c * LANES, LANES)]
                xbuf[pl.ds(c * LANES, LANES)] = (
                    lax.shift_right_logical(ids16, 1))
            return pltpu.async_copy(table_hbm.at[xbuf], gbuf, gsem)

        def out_store(i, buf, sem, do_wait):
            l, bt = block_lbt(i)
            for dt in range(dt_n):
                cp = pltpu.make_async_copy(
                    buf.at[pl.ds(dt * 8, 8)], out_hbm.at[l, dt, bt], sem)
                if do_wait:
                    cp.wait()
                else:
                    cp.start()

        def compute(i, gbuf, sbuf):
            l, bt = block_lbt(i)
            lr = l - t0

            @plsc.parallel_loop(0, BB, unroll=4)
            def _(t):
                for dq in range(nd2):
                    x = gbuf[t, dq * LANES:(dq + 1) * LANES]
                    plsc.store_scatter(tt_v, [sc_idx[dq] + t], x)

            qs = []
            for c in range(ncol):
                ids16 = ids_v[lr, pl.ds(bt * BB + c * LANES, LANES)]
                par = jnp.bitwise_and(ids16, jnp.int32(1))
                qs.append(par * jnp.int32(QOFF) + lane + jnp.int32(c * LANES))

            def pos_row(d):
                return pos_v[lr, d // 8, pl.ds((d % 8) * LANES, LANES)]

            def stats(d, carry):
                s, q = carry
                base = d * jnp.int32(STRIDE)
                pv = pos_row(d)
                s2, q2 = [], []
                for c in range(ncol):
                    v = plsc.load_gather(tt_v, [qs[c] + base]) + pv
                    s2.append(s[c] + v)
                    q2.append(q[c] + v * v)
                return s2, q2

            zero = [jnp.zeros((LANES,), jnp.float32)] * ncol
            s, q = plsc.parallel_loop(0, D, carry=(zero, zero), unroll=4)(
                stats)

            mv, rv = [], []
            for c in range(ncol):
                m = s[c] * inv_d
                var = q[c] * inv_d - m * m + jnp.float32(EPS)
                mv.append(m)
                rv.append(_rsqrt(var))

            @plsc.parallel_loop(0, D, unroll=4)
            def _(d):
                base = d * jnp.int32(STRIDE)
                pv = pos_row(d)
                gd = gb_v[0, d // 8, pl.ds((d % 8) * LANES, LANES)]
                bd = gb_v[1, d // 8, pl.ds((d % 8) * LANES, LANES)]
                for c in range(ncol):
                    v = plsc.load_gather(tt_v, [qs[c] + base]) + pv
                    sbuf[d, c * LANES:(c + 1) * LANES] = (
                        (v - mv[c]) * rv[c] * gd + bd)

        gather_start(0, g0_v, gsem0, x0_v)

        def pair_body(i2, _):
            for p in range(2):
                i = i2 * 2 + p
                gbuf, gsem = gbufs[p]
                sbuf, ssem = sbufs[p]

                @pl.when(i + 1 < bpw)
                def _():
                    gather_start(i + 1, gbufs[1 - p][0], gbufs[1 - p][1],
                                 xbufs[1 - p])

                pltpu.make_async_copy(
                    table_hbm.at[xbufs[p]], gbuf, gsem).wait()

                @pl.when(i >= 2)
                def _():
                    out_store(i - 2, sbuf, ssem, do_wait=True)

                compute(i, gbuf, sbuf)
                out_store(i, sbuf, ssem, do_wait=False)
            return 0

        lax.fori_loop(0, bpw // 2, pair_body, 0)
        for p in range(2):
            sbuf, ssem = sbufs[p]
            out_store(bpw - 2 + p, sbuf, ssem, do_wait=True)

    return sc_kernel


def kernel(input_ids, emb_table, pos_table, ln_gamma, ln_beta):
    B, L = input_ids.shape
    V, D = emb_table.shape
    T = B * L
    assert T % (NW * BB) == 0 and B % BB == 0 and 2 * D == BB
    assert V % 2 == 0 and (T // (NW * BB)) % 2 == 0

    ids2 = jnp.pad(jnp.transpose(input_ids, (1, 0)).astype(jnp.int32),
                   ((0, 8), (0, 0)))
    posp = jnp.pad(pos_table.astype(jnp.float32)[:L], ((0, 8), (0, 0)))
    posb = jnp.broadcast_to(
        posp[:, :, None], (L + 8, D, LANES)).reshape(L + 8, D // 8, BB)
    gb = jnp.stack([ln_gamma, ln_beta]).astype(jnp.float32)
    gbb = jnp.broadcast_to(
        gb[:, :, None], (2, D, LANES)).reshape(2, D // 8, BB)
    table2 = emb_table.reshape(V // 2, 2 * D)

    sc = _make_sc_kernel(B, L, D)
    out5 = sc(ids2, posb, gbb, table2)
    return out5.transpose(2, 4, 0, 1, 3).reshape(B, L, D)


# speedup vs baseline: 1.1418x; 1.1418x over previous
import functools

import jax
import jax.numpy as jnp
from jax import lax
from jax.experimental import pallas as pl
from jax.experimental.pallas import tpu as pltpu
from jax.experimental.pallas import tpu_sc as plsc

NC = 2
NS = 16
NW = NC * NS
LANES = 16
EPS = 1e-5
BB = 128
STRIDE = 129


def _rsqrt(v):
    i = lax.bitcast_convert_type(v, jnp.int32)
    i = jnp.int32(0x5F3759DF) - lax.shift_right_logical(i, 1)
    y = lax.bitcast_convert_type(i, jnp.float32)
    for _ in range(3):
        y = y * (1.5 - 0.5 * v * y * y)
    return y


def _make_sc_kernel(B, L, D):
    T = B * L
    n_blocks = T // BB
    bpw = n_blocks // NW
    nd2 = D // LANES
    ncol = BB // LANES
    dt_n = D // 8
    bt_n = B // BB

    mesh = plsc.VectorSubcoreMesh(core_axis_name="c", subcore_axis_name="s")

    @functools.partial(
        pl.kernel,
        out_type=jax.ShapeDtypeStruct((L, dt_n, bt_n, 8, BB), jnp.float32),
        mesh=mesh,
        compiler_params=pltpu.CompilerParams(
            needs_layout_passes=False, use_tc_tiling_on_sc=True),
        scratch_types=[
            pltpu.VMEM((16, B), jnp.int32),
            pltpu.VMEM((16, D // 8, BB), jnp.float32),
            pltpu.VMEM((2, D // 8, BB), jnp.float32),
            pltpu.VMEM((BB, 2 * D), jnp.float32),
            pltpu.VMEM((BB, 2 * D), jnp.float32),
            pltpu.VMEM((BB,), jnp.int32),
            pltpu.VMEM((BB,), jnp.int32),
            pltpu.VMEM((D * STRIDE,), jnp.float32),
            pltpu.VMEM((D, BB), jnp.float32),
            pltpu.VMEM((D, BB), jnp.float32),
            pltpu.SemaphoreType.DMA,
            pltpu.SemaphoreType.DMA,
            pltpu.SemaphoreType.DMA,
            pltpu.SemaphoreType.DMA,
        ],
    )
    def sc_kernel(ids_hbm, pos_hbm, gb_hbm, table_hbm, out_hbm,
                  ids_v, pos_v, gb_v, g0_v, g1_v, x0_v, x1_v, tt_v,
                  s0_v, s1_v, gsem0, gsem1, ssem0, ssem1):
        wid = lax.axis_index("s") * NC + lax.axis_index("c")
        g_base = wid * bpw
        t0 = (g_base // bt_n) // 8 * 8

        pltpu.sync_copy(ids_hbm.at[pl.ds(t0, 16)], ids_v)
        pltpu.sync_copy(pos_hbm.at[pl.ds(t0, 16)], pos_v)
        pltpu.sync_copy(gb_hbm, gb_v)

        lane = lax.iota(jnp.int32, LANES)
        inv_d = jnp.float32(1.0 / D)
        sc_idx = [(lane + dq * LANES) * jnp.int32(STRIDE)
                  for dq in range(nd2)]

        gbufs = [(g0_v, gsem0), (g1_v, gsem1)]
        xbufs = [x0_v, x1_v]
        sbufs = [(s0_v, ssem0), (s1_v, ssem1)]

        def block_lbt(i):
            g = g_base + i
            return g // bt_n, g % bt_n

        def gather_start(i, gbuf, gsem, xbuf):
            l, bt = block_lbt(i)
            lr = l - t0
            for c in range(ncol):
                ids16 = ids_v[lr, pl.ds(bt * BB + c * LANES, LANES)]
                xbuf[pl.ds(c * LANES, LANES)] = ids16
            return pltpu.async_copy(table_hbm.at[xbuf], gbuf, gsem)

        def out_store(i, buf, sem, do_wait):
            l, bt = block_lbt(i)
            for dt in range(dt_n):
                cp = pltpu.make_async_copy(
                    buf.at[pl.ds(dt * 8, 8)], out_hbm.at[l, dt, bt], sem)
                if do_wait:
                    cp.wait()
                else:
                    cp.start()

        def compute(i, gbuf, sbuf):
            l, bt = block_lbt(i)
            lr = l - t0

            @plsc.parallel_loop(0, BB, unroll=4)
            def _(t):
                for dq in range(nd2):
                    x = gbuf[t, dq * LANES:(dq + 1) * LANES]
                    plsc.store_scatter(tt_v, [sc_idx[dq] + t], x)

            qs = [lane + jnp.int32(c * LANES) for c in range(ncol)]

            def pos_row(d):
                return pos_v[lr, d // 8, pl.ds((d % 8) * LANES, LANES)]

            def stats(d, carry):
                s, q = carry
                base = d * jnp.int32(STRIDE)
                pv = pos_row(d)
                s2, q2 = [], []
                for c in range(ncol):
                    v = plsc.load_gather(tt_v, [qs[c] + base]) + pv
                    s2.append(s[c] + v)
                    q2.append(q[c] + v * v)
                return s2, q2

            zero = [jnp.zeros((LANES,), jnp.float32)] * ncol
            s, q = plsc.parallel_loop(0, D, carry=(zero, zero), unroll=4)(
                stats)

            mv, rv = [], []
            for c in range(ncol):
                m = s[c] * inv_d
                var = q[c] * inv_d - m * m + jnp.float32(EPS)
                mv.append(m)
                rv.append(_rsqrt(var))

            @plsc.parallel_loop(0, D, unroll=4)
            def _(d):
                base = d * jnp.int32(STRIDE)
                pv = pos_row(d)
                gd = gb_v[0, d // 8, pl.ds((d % 8) * LANES, LANES)]
                bd = gb_v[1, d // 8, pl.ds((d % 8) * LANES, LANES)]
                for c in range(ncol):
                    v = plsc.load_gather(tt_v, [qs[c] + base]) + pv
                    sbuf[d, c * LANES:(c + 1) * LANES] = (
                        (v - mv[c]) * rv[c] * gd + bd)

        gather_start(0, g0_v, gsem0, x0_v)

        def pair_body(i2, _):
            for p in range(2):
                i = i2 * 2 + p
                gbuf, gsem = gbufs[p]
                sbuf, ssem = sbufs[p]

                @pl.when(i + 1 < bpw)
                def _():
                    gather_start(i + 1, gbufs[1 - p][0], gbufs[1 - p][1],
                                 xbufs[1 - p])

                pltpu.make_async_copy(
                    table_hbm.at[xbufs[p]], gbuf, gsem).wait()

                @pl.when(i >= 2)
                def _():
                    out_store(i - 2, sbuf, ssem, do_wait=True)

                compute(i, gbuf, sbuf)
                out_store(i, sbuf, ssem, do_wait=False)
            return 0

        lax.fori_loop(0, bpw // 2, pair_body, 0)
        for p in range(2):
            sbuf, ssem = sbufs[p]
            out_store(bpw - 2 + p, sbuf, ssem, do_wait=True)

    return sc_kernel


def kernel(input_ids, emb_table, pos_table, ln_gamma, ln_beta):
    B, L = input_ids.shape
    V, D = emb_table.shape
    T = B * L
    assert T % (NW * BB) == 0 and B % BB == 0 and D % LANES == 0
    assert (T // (NW * BB)) % 2 == 0

    ids2 = jnp.pad(jnp.transpose(input_ids, (1, 0)).astype(jnp.int32),
                   ((0, 8), (0, 0)))
    table2 = jnp.pad(emb_table.astype(jnp.float32), ((0, 0), (0, D)))
    posp = jnp.pad(pos_table.astype(jnp.float32)[:L], ((0, 8), (0, 0)))
    posb = jnp.broadcast_to(
        posp[:, :, None], (L + 8, D, LANES)).reshape(L + 8, D // 8, BB)
    gb = jnp.stack([ln_gamma, ln_beta]).astype(jnp.float32)
    gbb = jnp.broadcast_to(
        gb[:, :, None], (2, D, LANES)).reshape(2, D // 8, BB)

    sc = _make_sc_kernel(B, L, D)
    out5 = sc(ids2, posb, gbb, table2)
    return out5.transpose(2, 4, 0, 1, 3).reshape(B, L, D)
